# baseline (speedup 1.0000x reference)
import jax
import jax.numpy as jnp
from jax.experimental import pallas as pl


def _copy_body(x_ref, o_ref):
    o_ref[...] = x_ref[...]


def _pallas_copy(x):
    g = 50
    blk = (x.shape[0] // g,) + x.shape[1:]
    idx = lambda i: (i,) + (0,) * (x.ndim - 1)
    return pl.pallas_call(
        _copy_body,
        out_shape=jax.ShapeDtypeStruct(x.shape, x.dtype),
        grid=(g,),
        in_specs=[pl.BlockSpec(blk, idx)],
        out_specs=pl.BlockSpec(blk, idx),
    )(x)


def kernel(nodes, edges, edge_index):
    b, n, f = nodes.shape
    m = edge_index.shape[1]
    offs = (jnp.arange(b, dtype=edge_index.dtype) * n)[:, None, None]
    indexlist = (edge_index + offs).reshape(b * m, 2)
    valuelist = edges.reshape(b * m)
    order = jnp.argsort(indexlist[:, 1], stable=True)
    indexlist = jnp.take(indexlist, order, axis=0)
    valuelist = jnp.take(valuelist, order, axis=0)
    order = jnp.argsort(indexlist[:, 0], stable=True)
    indexlist = jnp.take(indexlist, order, axis=0)
    valuelist = jnp.take(valuelist, order, axis=0)
    indexlist = _pallas_copy(indexlist.reshape(-1)).reshape(b * m, 2).astype(jnp.int64)
    valuelist = _pallas_copy(valuelist)
    dense_shape = jnp.array([b * n, b * n], dtype=jnp.int64)
    return indexlist, valuelist, dense_shape


# speedup vs baseline: 2.1521x; 2.1521x over previous
import functools

import jax
import jax.numpy as jnp
from jax import lax
from jax.experimental import pallas as pl
from jax.experimental.pallas import tpu as pltpu, tpu_sc as plsc

T = 16
NC = 2
K = 1024


def _radix_body(meta, ei_hbm, ev_hbm, oidx_hbm, oval_hbm,
                ei_v, val_v, keys_v, dst2_v, hist_v, off_v, grid_v, outb_v,
                grid_sh, bufa_k, bufa_v, bufb_k, bufb_v, sem):
    B, N, M = meta
    E = M // T
    EP = ((E + 127) // 128) * 128
    VPB = E // 16
    BPC = B // NC
    c = lax.axis_index("c")
    t = lax.axis_index("s")
    iota = lax.iota(jnp.int32, 16)
    ones = jnp.ones((16,), jnp.int32)
    zeros = jnp.zeros((16,), jnp.int32)

    for z in range((EP - E) // 16):
        row, col = (E + z * 16) // 128, (E + z * 16) % 128
        dst2_v[row, pl.ds(col, 16)] = M + z * 16 + iota

    def zero_hist():
        def zb(z, carry):
            hist_v[pl.ds(z * 16, 16)] = zeros
            return carry
        lax.fori_loop(0, K // 16, zb, 0)

    def scan_offsets():
        pltpu.sync_copy(grid_sh, grid_v)

        def zb(z, carry):
            col = zeros
            pre = zeros
            for tt in range(T):
                h = grid_v[pl.ds(tt * K + z * 16, 16)]
                pre = pre + h * (t > tt).astype(jnp.int32)
                col = col + h
            incl = plsc.cumsum(col)
            off_v[pl.ds(z * 16, 16)] = incl - col + pre + carry
            return carry + jnp.sum(col)

        lax.fori_loop(0, K // 16, zb, jnp.int32(0))

    def rank_pass(shift):
        def rb(i, carry):
            kk = keys_v[pl.ds(i * 16, 16)]
            if shift:
                d = lax.shift_right_logical(kk, shift)
            else:
                d = lax.bitwise_and(kk, K - 1)
            cnt, _ = plsc.scan_count(d)
            base = plsc.load_gather(off_v, [d])
            row = lax.shift_right_logical(i, 3)
            col = lax.bitwise_and(i, 7) * 16
            dst2_v[row, pl.ds(col, 16)] = base + cnt - 1
            plsc.addupdate_scatter(off_v, [d], ones)
            return carry
        lax.fori_loop(0, VPB, rb, 0)

    def scatter_to(buf_k, buf_v):
        rows = EP // 128
        group = 8
        for r0 in range(0, rows, group):
            copies = []
            for row in range(r0, min(r0 + group, rows)):
                copies.append(pltpu.async_copy(
                    keys_v.at[pl.ds(row * 128, 128)],
                    buf_k.at[dst2_v.at[row]], sem))
                copies.append(pltpu.async_copy(
                    val_v.at[pl.ds(row * 128, 128)],
                    buf_v.at[dst2_v.at[row]], sem))
            for cp in copies:
                cp.wait()

    def batch(j, carry):
        b = c * BPC + j
        pltpu.sync_copy(ei_hbm.at[pl.ds(b * 2 * M + t * 2 * E, 2 * E)], ei_v)
        pltpu.sync_copy(ev_hbm.at[pl.ds(b * M + t * E, E)], val_v.at[pl.ds(0, E)])

        zero_hist()

        def p1(i, cc):
            pos = iota * 2 + i * 32
            e0 = plsc.load_gather(ei_v, [pos])
            e1 = plsc.load_gather(ei_v, [pos + 1])
            keys_v[pl.ds(i * 16, 16)] = e0 * K + e1
            plsc.addupdate_scatter(hist_v, [e1], ones)
            return cc
        lax.fori_loop(0, VPB, p1, 0)
        pltpu.sync_copy(hist_v, grid_sh.at[pl.ds(t * K, K)])
        plsc.subcore_barrier()
        scan_offsets()
        rank_pass(0)
        scatter_to(bufa_k, bufa_v)
        plsc.subcore_barrier()

        pltpu.sync_copy(bufa_k.at[pl.ds(t * E, E)], keys_v.at[pl.ds(0, E)])
        pltpu.sync_copy(bufa_v.at[pl.ds(t * E, E)], val_v.at[pl.ds(0, E)])
        zero_hist()

        def p2(i, cc):
            kk = keys_v[pl.ds(i * 16, 16)]
            plsc.addupdate_scatter(
                hist_v, [lax.shift_right_logical(kk, 10)], ones)
            return cc
        lax.fori_loop(0, VPB, p2, 0)
        pltpu.sync_copy(hist_v, grid_sh.at[pl.ds(t * K, K)])
        plsc.subcore_barrier()
        scan_offsets()
        rank_pass(10)
        scatter_to(bufb_k, bufb_v)
        plsc.subcore_barrier()

        pltpu.sync_copy(bufb_k.at[pl.ds(t * E, E)], keys_v.at[pl.ds(0, E)])
        pltpu.sync_copy(bufb_v.at[pl.ds(t * E, E)], val_v.at[pl.ds(0, E)])
        base_node = b * N

        def fo(i, cc):
            kk = keys_v[pl.ds(i * 16, 16)]
            e0 = lax.shift_right_logical(kk, 10)
            e1 = lax.bitwise_and(kk, K - 1)
            pos = iota * 2 + i * 32
            plsc.store_scatter(outb_v, [pos], e0 + base_node)
            plsc.store_scatter(outb_v, [pos + 1], e1 + base_node)
            return cc
        lax.fori_loop(0, VPB, fo, 0)
        pltpu.sync_copy(outb_v, oidx_hbm.at[pl.ds((b * M + t * E) * 2, 2 * E)])
        pltpu.sync_copy(val_v.at[pl.ds(0, E)],
                        oval_hbm.at[pl.ds(b * M + t * E, E)])
        return carry

    lax.fori_loop(0, BPC, batch, 0)


def kernel(nodes, edges, edge_index):
    b, n, f = nodes.shape
    m = edge_index.shape[1]
    e = m // T
    ep = ((e + 127) // 128) * 128
    ei = edge_index.reshape(b * m * 2)
    ev = edges.reshape(b * m)
    mesh = plsc.VectorSubcoreMesh(core_axis_name="c", subcore_axis_name="s")
    fn = pl.kernel(
        functools.partial(_radix_body, (b, n, m)),
        out_type=(jax.ShapeDtypeStruct((b * m * 2,), jnp.int32),
                  jax.ShapeDtypeStruct((b * m,), jnp.float32)),
        mesh=mesh,
        compiler_params=pltpu.CompilerParams(needs_layout_passes=False),
        scratch_types=[
            pltpu.VMEM((2 * e,), jnp.int32),
            pltpu.VMEM((ep,), jnp.float32),
            pltpu.VMEM((ep,), jnp.int32),
            pltpu.VMEM((ep // 128, 128), jnp.int32),
            pltpu.VMEM((K,), jnp.int32),
            pltpu.VMEM((K,), jnp.int32),
            pltpu.VMEM((T * K,), jnp.int32),
            pltpu.VMEM((2 * e,), jnp.int32),
            pltpu.VMEM_SHARED((T * K,), jnp.int32),
            pltpu.VMEM_SHARED((m + 128,), jnp.int32),
            pltpu.VMEM_SHARED((m + 128,), jnp.float32),
            pltpu.VMEM_SHARED((m + 128,), jnp.int32),
            pltpu.VMEM_SHARED((m + 128,), jnp.float32),
            pltpu.SemaphoreType.DMA,
        ],
    )
    oidx, oval = fn(ei, ev)
    indexlist = oidx.reshape(b * m, 2).astype(jnp.int64)
    dense_shape = jnp.array([b * n, b * n], dtype=jnp.int64)
    return indexlist, oval, dense_shape


# speedup vs baseline: 21.3889x; 9.9387x over previous
import functools

import jax
import jax.numpy as jnp
from jax import lax
from jax.experimental import pallas as pl
from jax.experimental.pallas import tpu as pltpu, tpu_sc as plsc

T = 16
NC = 2
K = 1024


def _radix_body(meta, e0_hbm, e1_hbm, ev_hbm, oc0_hbm, oc1_hbm, oval_hbm,
                e0_v, e1_v, val_v, keys_v, dst2_v, hist_v, off_v, grid_v,
                grid_sh, bufa_k, bufa_v, bufb_k, bufb_v, sem):
    B, N, M = meta
    E = M // T
    EP = ((E + 127) // 128) * 128
    VPB = E // 16
    BPC = B // NC
    c = lax.axis_index("c")
    t = lax.axis_index("s")
    iota = lax.iota(jnp.int32, 16)
    ones = jnp.ones((16,), jnp.int32)
    zeros = jnp.zeros((16,), jnp.int32)

    for z in range((EP - E) // 16):
        row, col = (E + z * 16) // 128, (E + z * 16) % 128
        dst2_v[row, pl.ds(col, 16)] = M + z * 16 + iota

    def zero_hist():
        def zb(z, carry):
            hist_v[pl.ds(z * 16, 16)] = zeros
            return carry
        lax.fori_loop(0, K // 16, zb, 0)

    def scan_offsets():
        pltpu.sync_copy(grid_sh, grid_v)

        def zb(z, carry):
            col = zeros
            pre = zeros
            for tt in range(T):
                h = grid_v[pl.ds(tt * K + z * 16, 16)]
                pre = pre + h * (t > tt).astype(jnp.int32)
                col = col + h
            incl = plsc.cumsum(col)
            off_v[pl.ds(z * 16, 16)] = incl - col + pre + carry
            return carry + jnp.sum(col)

        lax.fori_loop(0, K // 16, zb, jnp.int32(0))

    def rank_pass(shift):
        def rb(i, carry):
            kk = keys_v[pl.ds(i * 16, 16)]
            if shift:
                d = lax.shift_right_logical(kk, shift)
            else:
                d = lax.bitwise_and(kk, K - 1)
            cnt, _ = plsc.scan_count(d)
            base = plsc.load_gather(off_v, [d])
            row = lax.shift_right_logical(i, 3)
            col = lax.bitwise_and(i, 7) * 16
            dst2_v[row, pl.ds(col, 16)] = base + cnt - 1
            plsc.addupdate_scatter(off_v, [d], ones)
            return carry
        lax.fori_loop(0, VPB, rb, 0)

    def scatter_to(buf_k, buf_v):
        rows = EP // 128
        group = 8
        for r0 in range(0, rows, group):
            copies = []
            for row in range(r0, min(r0 + group, rows)):
                copies.append(pltpu.async_copy(
                    keys_v.at[pl.ds(row * 128, 128)],
                    buf_k.at[dst2_v.at[row]], sem))
                copies.append(pltpu.async_copy(
                    val_v.at[pl.ds(row * 128, 128)],
                    buf_v.at[dst2_v.at[row]], sem))
            for cp in copies:
                cp.wait()

    def batch(j, carry):
        b = c * BPC + j
        base = b * M + t * E
        pltpu.sync_copy(e0_hbm.at[pl.ds(base, E)], e0_v)
        pltpu.sync_copy(e1_hbm.at[pl.ds(base, E)], e1_v)
        pltpu.sync_copy(ev_hbm.at[pl.ds(base, E)], val_v.at[pl.ds(0, E)])

        zero_hist()

        def p1(i, cc):
            e0 = e0_v[pl.ds(i * 16, 16)]
            e1 = e1_v[pl.ds(i * 16, 16)]
            keys_v[pl.ds(i * 16, 16)] = e0 * K + e1
            plsc.addupdate_scatter(hist_v, [e1], ones)
            return cc
        lax.fori_loop(0, VPB, p1, 0)
        pltpu.sync_copy(hist_v, grid_sh.at[pl.ds(t * K, K)])
        plsc.subcore_barrier()
        scan_offsets()
        rank_pass(0)
        scatter_to(bufa_k, bufa_v)
        plsc.subcore_barrier()

        pltpu.sync_copy(bufa_k.at[pl.ds(t * E, E)], keys_v.at[pl.ds(0, E)])
        pltpu.sync_copy(bufa_v.at[pl.ds(t * E, E)], val_v.at[pl.ds(0, E)])
        zero_hist()

        def p2(i, cc):
            kk = keys_v[pl.ds(i * 16, 16)]
            plsc.addupdate_scatter(
                hist_v, [lax.shift_right_logical(kk, 10)], ones)
            return cc
        lax.fori_loop(0, VPB, p2, 0)
        pltpu.sync_copy(hist_v, grid_sh.at[pl.ds(t * K, K)])
        plsc.subcore_barrier()
        scan_offsets()
        rank_pass(10)
        scatter_to(bufb_k, bufb_v)
        plsc.subcore_barrier()

        pltpu.sync_copy(bufb_k.at[pl.ds(t * E, E)], keys_v.at[pl.ds(0, E)])
        pltpu.sync_copy(bufb_v.at[pl.ds(t * E, E)], val_v.at[pl.ds(0, E)])
        base_node = b * N

        def fo(i, cc):
            kk = keys_v[pl.ds(i * 16, 16)]
            e0_v[pl.ds(i * 16, 16)] = \
                lax.shift_right_logical(kk, 10) + base_node
            e1_v[pl.ds(i * 16, 16)] = \
                lax.bitwise_and(kk, K - 1) + base_node
            return cc
        lax.fori_loop(0, VPB, fo, 0)
        pltpu.sync_copy(e0_v, oc0_hbm.at[pl.ds(base, E)])
        pltpu.sync_copy(e1_v, oc1_hbm.at[pl.ds(base, E)])
        pltpu.sync_copy(val_v.at[pl.ds(0, E)], oval_hbm.at[pl.ds(base, E)])
        return carry

    lax.fori_loop(0, BPC, batch, 0)


def kernel(nodes, edges, edge_index):
    b, n, f = nodes.shape
    m = edge_index.shape[1]
    e = m // T
    ep = ((e + 127) // 128) * 128
    e0f = edge_index[:, :, 0].reshape(b * m)
    e1f = edge_index[:, :, 1].reshape(b * m)
    ev = edges.reshape(b * m)
    mesh = plsc.VectorSubcoreMesh(core_axis_name="c", subcore_axis_name="s")
    fn = pl.kernel(
        functools.partial(_radix_body, (b, n, m)),
        out_type=(jax.ShapeDtypeStruct((b * m,), jnp.int32),
                  jax.ShapeDtypeStruct((b * m,), jnp.int32),
                  jax.ShapeDtypeStruct((b * m,), jnp.float32)),
        mesh=mesh,
        compiler_params=pltpu.CompilerParams(needs_layout_passes=False),
        scratch_types=[
            pltpu.VMEM((e,), jnp.int32),
            pltpu.VMEM((e,), jnp.int32),
            pltpu.VMEM((ep,), jnp.float32),
            pltpu.VMEM((ep,), jnp.int32),
            pltpu.VMEM((ep // 128, 128), jnp.int32),
            pltpu.VMEM((K,), jnp.int32),
            pltpu.VMEM((K,), jnp.int32),
            pltpu.VMEM((T * K,), jnp.int32),
            pltpu.VMEM_SHARED((T * K,), jnp.int32),
            pltpu.VMEM_SHARED((m + 128,), jnp.int32),
            pltpu.VMEM_SHARED((m + 128,), jnp.float32),
            pltpu.VMEM_SHARED((m + 128,), jnp.int32),
            pltpu.VMEM_SHARED((m + 128,), jnp.float32),
            pltpu.SemaphoreType.DMA,
        ],
    )
    oc0, oc1, oval = fn(e0f, e1f, ev)
    indexlist = jnp.stack([oc0, oc1], axis=1).astype(jnp.int64)
    dense_shape = jnp.array([b * n, b * n], dtype=jnp.int64)
    return indexlist, oval, dense_shape


# speedup vs baseline: 29.5568x; 1.3819x over previous
import functools

import jax
import jax.numpy as jnp
from jax import lax
from jax.experimental import pallas as pl
from jax.experimental.pallas import tpu as pltpu, tpu_sc as plsc

T = 16
NC = 2
K = 1024


def _radix_body(meta, e0_hbm, e1_hbm, ev_hbm, oc0_hbm, oc1_hbm, oval_hbm,
                e0_v, e1_v, val_v, keys_v, dst2_v, hista_v, histb_v, hist_v,
                offa_v, offb_v, grid_v,
                grid_sh, bufa_k, bufa_v, bufb_k, bufb_v, sem):
    B, N, M = meta
    E = M // T
    H = E // 2
    EP = ((E + 127) // 128) * 128
    VPH = H // 16
    BPC = B // NC
    c = lax.axis_index("c")
    t = lax.axis_index("s")
    iota = lax.iota(jnp.int32, 16)
    ones = jnp.ones((16,), jnp.int32)
    zeros = jnp.zeros((16,), jnp.int32)

    for z in range((EP - E) // 16):
        row, col = (E + z * 16) // 128, (E + z * 16) % 128
        dst2_v[row, pl.ds(col, 16)] = M + z * 16 + iota

    def dst_store(flat, vec):
        row = lax.shift_right_logical(flat, 7)
        col = lax.bitwise_and(flat, 127)
        dst2_v[row, pl.ds(col, 16)] = vec

    def zero_hists():
        @plsc.parallel_loop(0, K // 16, unroll=4)
        def _(z):
            hista_v[pl.ds(z * 16, 16)] = zeros
            histb_v[pl.ds(z * 16, 16)] = zeros

    def merge_hists():
        @plsc.parallel_loop(0, K // 16, unroll=4)
        def _(z):
            hist_v[pl.ds(z * 16, 16)] = (
                hista_v[pl.ds(z * 16, 16)] + histb_v[pl.ds(z * 16, 16)])

    def scan_offsets():
        pltpu.sync_copy(grid_sh, grid_v)

        @plsc.parallel_loop(0, K // 16, unroll=2, carry=jnp.int32(0))
        def _(z, carry):
            col = zeros
            pre = zeros
            for tt in range(T):
                h = grid_v[pl.ds(tt * K + z * 16, 16)]
                pre = pre + h * (t > tt).astype(jnp.int32)
                col = col + h
            incl = plsc.cumsum(col)
            offa = incl - col + pre + carry
            offa_v[pl.ds(z * 16, 16)] = offa
            offb_v[pl.ds(z * 16, 16)] = offa + hista_v[pl.ds(z * 16, 16)]
            return carry + jnp.sum(col)

    def rank_pass(shift):
        def digit(kk):
            if shift:
                return lax.shift_right_logical(kk, shift)
            return lax.bitwise_and(kk, K - 1)

        def rb(i, carry):
            ka = keys_v[pl.ds(i * 16, 16)]
            kb = keys_v[pl.ds(H + i * 16, 16)]
            da = digit(ka)
            db = digit(kb)
            ca, _ = plsc.scan_count(da)
            cb, _ = plsc.scan_count(db)
            basea = plsc.load_gather(offa_v, [da])
            baseb = plsc.load_gather(offb_v, [db])
            dst_store(i * 16, basea + ca - 1)
            dst_store(H + i * 16, baseb + cb - 1)
            plsc.addupdate_scatter(offa_v, [da], ones)
            plsc.addupdate_scatter(offb_v, [db], ones)
            return carry
        lax.fori_loop(0, VPH, rb, 0)

    def scatter_to(buf_k, buf_v):
        copies = []
        for row in range(EP // 128):
            copies.append(pltpu.async_copy(
                keys_v.at[pl.ds(row * 128, 128)],
                buf_k.at[dst2_v.at[row]], sem))
            copies.append(pltpu.async_copy(
                val_v.at[pl.ds(row * 128, 128)],
                buf_v.at[dst2_v.at[row]], sem))
        for cp in copies:
            cp.wait()

    def batch(j, carry):
        b = c * BPC + j
        base = b * M + t * E
        pltpu.sync_copy(e0_hbm.at[pl.ds(base, E)], e0_v)
        pltpu.sync_copy(e1_hbm.at[pl.ds(base, E)], e1_v)
        pltpu.sync_copy(ev_hbm.at[pl.ds(base, E)], val_v.at[pl.ds(0, E)])

        zero_hists()

        @plsc.parallel_loop(0, VPH, unroll=2)
        def _(i):
            e0a = e0_v[pl.ds(i * 16, 16)]
            e1a = e1_v[pl.ds(i * 16, 16)]
            e0b = e0_v[pl.ds(H + i * 16, 16)]
            e1b = e1_v[pl.ds(H + i * 16, 16)]
            keys_v[pl.ds(i * 16, 16)] = e0a * K + e1a
            keys_v[pl.ds(H + i * 16, 16)] = e0b * K + e1b
            plsc.addupdate_scatter(hista_v, [e1a], ones)
            plsc.addupdate_scatter(histb_v, [e1b], ones)
        merge_hists()
        pltpu.sync_copy(hist_v, grid_sh.at[pl.ds(t * K, K)])
        plsc.subcore_barrier()
        scan_offsets()
        rank_pass(0)
        scatter_to(bufa_k, bufa_v)
        plsc.subcore_barrier()

        pltpu.sync_copy(bufa_k.at[pl.ds(t * E, E)], keys_v.at[pl.ds(0, E)])
        pltpu.sync_copy(bufa_v.at[pl.ds(t * E, E)], val_v.at[pl.ds(0, E)])
        zero_hists()

        @plsc.parallel_loop(0, VPH, unroll=2)
        def _(i):
            ka = keys_v[pl.ds(i * 16, 16)]
            kb = keys_v[pl.ds(H + i * 16, 16)]
            plsc.addupdate_scatter(
                hista_v, [lax.shift_right_logical(ka, 10)], ones)
            plsc.addupdate_scatter(
                histb_v, [lax.shift_right_logical(kb, 10)], ones)
        merge_hists()
        pltpu.sync_copy(hist_v, grid_sh.at[pl.ds(t * K, K)])
        plsc.subcore_barrier()
        scan_offsets()
        rank_pass(10)
        scatter_to(bufb_k, bufb_v)
        plsc.subcore_barrier()

        pltpu.sync_copy(bufb_k.at[pl.ds(t * E, E)], keys_v.at[pl.ds(0, E)])
        pltpu.sync_copy(bufb_v.at[pl.ds(t * E, E)], val_v.at[pl.ds(0, E)])
        base_node = b * N

        @plsc.parallel_loop(0, 2 * VPH, unroll=4)
        def _(i):
            kk = keys_v[pl.ds(i * 16, 16)]
            e0_v[pl.ds(i * 16, 16)] = \
                lax.shift_right_logical(kk, 10) + base_node
            e1_v[pl.ds(i * 16, 16)] = \
                lax.bitwise_and(kk, K - 1) + base_node
        pltpu.sync_copy(e0_v, oc0_hbm.at[pl.ds(base, E)])
        pltpu.sync_copy(e1_v, oc1_hbm.at[pl.ds(base, E)])
        pltpu.sync_copy(val_v.at[pl.ds(0, E)], oval_hbm.at[pl.ds(base, E)])
        return carry

    lax.fori_loop(0, BPC, batch, 0)


def kernel(nodes, edges, edge_index):
    b, n, f = nodes.shape
    m = edge_index.shape[1]
    e = m // T
    ep = ((e + 127) // 128) * 128
    e0f = edge_index[:, :, 0].reshape(b * m)
    e1f = edge_index[:, :, 1].reshape(b * m)
    ev = edges.reshape(b * m)
    mesh = plsc.VectorSubcoreMesh(core_axis_name="c", subcore_axis_name="s")
    fn = pl.kernel(
        functools.partial(_radix_body, (b, n, m)),
        out_type=(jax.ShapeDtypeStruct((b * m,), jnp.int32),
                  jax.ShapeDtypeStruct((b * m,), jnp.int32),
                  jax.ShapeDtypeStruct((b * m,), jnp.float32)),
        mesh=mesh,
        compiler_params=pltpu.CompilerParams(needs_layout_passes=False),
        scratch_types=[
            pltpu.VMEM((e,), jnp.int32),
            pltpu.VMEM((e,), jnp.int32),
            pltpu.VMEM((ep,), jnp.float32),
            pltpu.VMEM((ep,), jnp.int32),
            pltpu.VMEM((ep // 128, 128), jnp.int32),
            pltpu.VMEM((K,), jnp.int32),
            pltpu.VMEM((K,), jnp.int32),
            pltpu.VMEM((K,), jnp.int32),
            pltpu.VMEM((K,), jnp.int32),
            pltpu.VMEM((K,), jnp.int32),
            pltpu.VMEM((T * K,), jnp.int32),
            pltpu.VMEM_SHARED((T * K,), jnp.int32),
            pltpu.VMEM_SHARED((m + 128,), jnp.int32),
            pltpu.VMEM_SHARED((m + 128,), jnp.float32),
            pltpu.VMEM_SHARED((m + 128,), jnp.int32),
            pltpu.VMEM_SHARED((m + 128,), jnp.float32),
            pltpu.SemaphoreType.DMA,
        ],
    )
    oc0, oc1, oval = fn(e0f, e1f, ev)
    indexlist = jnp.stack([oc0, oc1], axis=1).astype(jnp.int64)
    dense_shape = jnp.array([b * n, b * n], dtype=jnp.int64)
    return indexlist, oval, dense_shape
